# baseline (speedup 1.0000x reference)
import functools

import jax
import jax.numpy as jnp
from jax import lax
from jax.experimental import pallas as pl
from jax.experimental.pallas import tpu as pltpu
from jax.experimental.pallas import tpu_sc as plsc

_BETA = 0.25
_TILE = 1024


def _split3(x):
    hi = x.astype(jnp.bfloat16).astype(jnp.float32)
    r = x - hi
    mid = r.astype(jnp.bfloat16).astype(jnp.float32)
    return hi, mid, r - mid


def _argmin_tile(z, a_ref):
    z2 = jnp.sum(z * z, axis=1, keepdims=True)
    z2h, z2m, z2l = _split3(z2)
    ones = jnp.ones((z.shape[0], 3), jnp.float32)
    z_aug = jnp.concatenate([z, z2h, z2m, z2l, ones], axis=1)
    d2 = jax.lax.dot_general(
        z_aug, a_ref[...], (((1,), (0,)), ((), ())),
        preferred_element_type=jnp.float32,
        precision=jax.lax.Precision.DEFAULT)
    return jnp.argmin(d2, axis=1).astype(jnp.int32)


def _vq_body_a(z_ref, cb_ref, w_ref, idx_ref, eff_ref, a_ref):
    i = pl.program_id(0)

    @pl.when(i == 0)
    def _init():
        eff = jax.lax.dot_general(
            cb_ref[...], w_ref[...], (((1,), (1,)), ((), ())),
            preferred_element_type=jnp.float32,
            precision=jax.lax.Precision.DEFAULT)
        eff_ref[:, 0:eff.shape[1]] = eff
        effT = jnp.swapaxes(eff, 0, 1)
        k = effT.shape[1]
        c2 = jnp.sum(effT * effT, axis=0, keepdims=True)
        c2h, c2m, c2l = _split3(c2)
        a_ref[...] = jnp.concatenate(
            [-2.0 * effT, jnp.ones((3, k), jnp.float32), c2h, c2m, c2l],
            axis=0)

    idx_ref[i, 0, :] = _argmin_tile(z_ref[0], a_ref)


def _vq_body_b(z_ref, a_ref, idx_ref):
    idx_ref[pl.program_id(0), 0, :] = _argmin_tile(z_ref[0], a_ref)


def _distance_argmin_a(z_e, cb, W, n_steps):
    B, T, D = z_e.shape
    K = cb.shape[0]
    t_per = T // _TILE
    return pl.pallas_call(
        _vq_body_a,
        grid=(n_steps,),
        in_specs=[
            pl.BlockSpec((1, _TILE, D), lambda i: (i // t_per, i % t_per, 0)),
            pl.BlockSpec((K, D), lambda i: (0, 0)),
            pl.BlockSpec((D, D), lambda i: (0, 0)),
        ],
        out_specs=[
            pl.BlockSpec((n_steps, 1, _TILE), lambda i: (0, 0, 0)),
            pl.BlockSpec((K, 128), lambda i: (0, 0)),
            pl.BlockSpec((D + 6, K), lambda i: (0, 0)),
        ],
        out_shape=[
            jax.ShapeDtypeStruct((n_steps, 1, _TILE), jnp.int32),
            jax.ShapeDtypeStruct((K, 128), jnp.float32),
            jax.ShapeDtypeStruct((D + 6, K), jnp.float32),
        ],
    )(z_e, cb, W)


def _distance_argmin_b(z_e, a, off, n_steps):
    B, T, D = z_e.shape
    A, K = a.shape
    t_per = T // _TILE
    return pl.pallas_call(
        _vq_body_b,
        grid=(n_steps,),
        in_specs=[
            pl.BlockSpec((1, _TILE, D),
                         lambda i: ((i + off) // t_per, (i + off) % t_per, 0)),
            pl.BlockSpec((A, K), lambda i: (0, 0)),
        ],
        out_specs=[
            pl.BlockSpec((n_steps, 1, _TILE), lambda i: (0, 0, 0)),
        ],
        out_shape=[
            jax.ShapeDtypeStruct((n_steps, 1, _TILE), jnp.int32),
        ],
    )(z_e, a)


def _epilogue_body(z_ref, zq1_ref, zq2_ref, idx1_ref, idx2_ref,
                   zq_ref, loss_ref, ci_ref, *, inv_nd):
    z = z_ref[...]
    b, t, d = z.shape
    zq1 = zq1_ref[:, 0:d].reshape(b // 2, t, d)
    zq2 = zq2_ref[:, 0:d].reshape(b // 2, t, d)
    zq = jnp.concatenate([zq1, zq2], axis=0)
    zq_ref[...] = zq
    diff = z - zq
    loss_ref[...] = (jnp.sum(diff * diff) * ((1.0 + _BETA) * inv_nd)
                     ).reshape(1, 1)
    ci_ref[...] = jnp.concatenate(
        [idx1_ref[...].reshape(b // 2, t), idx2_ref[...].reshape(b // 2, t)],
        axis=0)


def _slice_and_loss(z_e, zq_pad1, zq_pad2, idx1, idx2):
    B, T, D = z_e.shape
    body = functools.partial(_epilogue_body, inv_nd=1.0 / (B * T * D))
    return pl.pallas_call(
        body,
        out_shape=[
            jax.ShapeDtypeStruct((B, T, D), jnp.float32),
            jax.ShapeDtypeStruct((1, 1), jnp.float32),
            jax.ShapeDtypeStruct((B, T), jnp.int32),
        ],
    )(z_e, zq_pad1, zq_pad2, idx1, idx2)


def _sc_gather(eff, idx3):
    K, D = eff.shape
    n_steps, _, tile = idx3.shape
    N = n_steps * tile
    info = plsc.get_sparse_core_info()
    nw = info.num_cores * info.num_subcores
    per_w = N // nw
    w_per_step = tile // per_w
    mesh = plsc.VectorSubcoreMesh(core_axis_name="c", subcore_axis_name="s")

    @functools.partial(
        pl.kernel, mesh=mesh,
        out_type=jax.ShapeDtypeStruct((N, D), jnp.float32),
        scratch_types=[
            pltpu.VMEM((per_w,), jnp.int32),
            pltpu.VMEM((per_w, D), jnp.float32),
            pltpu.SemaphoreType.DMA,
        ],
    )
    def gather(table_hbm, idx_hbm, out_hbm, idx_v, rows_v, sem):
        wid = lax.axis_index("s") * info.num_cores + lax.axis_index("c")
        step = wid // w_per_step
        col = (wid % w_per_step) * per_w
        pltpu.sync_copy(idx_hbm.at[step, 0, pl.ds(col, per_w)], idx_v)
        pltpu.async_copy(table_hbm.at[idx_v], rows_v, sem).wait()
        pltpu.sync_copy(rows_v, out_hbm.at[pl.ds(wid * per_w, per_w)])

    return gather(eff, idx3)


def kernel(z_e, codebook, W):
    B, T, D = z_e.shape
    n_steps = (B * T) // _TILE
    n1 = n_steps // 2
    idx1, eff_pad, a = _distance_argmin_a(z_e, codebook, W, n1)
    zq_pad1 = _sc_gather(eff_pad, idx1)
    idx2, = _distance_argmin_b(z_e, a, n1, n_steps - n1)
    zq_pad2 = _sc_gather(eff_pad, idx2)
    z_q, loss, code_indices = _slice_and_loss(z_e, zq_pad1, zq_pad2,
                                              idx1, idx2)
    return z_q, loss[0, 0], code_indices


# speedup vs baseline: 1.0871x; 1.0871x over previous
import functools

import jax
import jax.numpy as jnp
from jax import lax
from jax.experimental import pallas as pl
from jax.experimental.pallas import tpu as pltpu
from jax.experimental.pallas import tpu_sc as plsc

_BETA = 0.25
_TILE = 1024


def _split3(x):
    hi = x.astype(jnp.bfloat16).astype(jnp.float32)
    r = x - hi
    mid = r.astype(jnp.bfloat16).astype(jnp.float32)
    return hi, mid, r - mid


def _vq_body(z_ref, cb_ref, w_ref, idx_ref, eff_ref, a_ref):
    i = pl.program_id(0)

    @pl.when(i == 0)
    def _init():
        eff = jax.lax.dot_general(
            cb_ref[...], w_ref[...], (((1,), (1,)), ((), ())),
            preferred_element_type=jnp.float32,
            precision=jax.lax.Precision.DEFAULT)
        eff_ref[:, 0:eff.shape[1]] = eff
        effT = jnp.swapaxes(eff, 0, 1)
        k = effT.shape[1]
        c2 = jnp.sum(effT * effT, axis=0, keepdims=True)
        c2h, c2m, c2l = _split3(c2)
        a_ref[...] = jnp.concatenate(
            [-2.0 * effT, jnp.ones((3, k), jnp.float32), c2h, c2m, c2l],
            axis=0)

    z = z_ref[0]
    z2 = jnp.sum(z * z, axis=1, keepdims=True)
    z2h, z2m, z2l = _split3(z2)
    ones = jnp.ones((z.shape[0], 3), jnp.float32)
    z_aug = jnp.concatenate([z, z2h, z2m, z2l, ones], axis=1)
    d2 = jax.lax.dot_general(
        z_aug, a_ref[...], (((1,), (0,)), ((), ())),
        preferred_element_type=jnp.float32,
        precision=jax.lax.Precision.DEFAULT)
    idx_ref[i, 0, :] = jnp.argmin(d2, axis=1).astype(jnp.int32)


def _distance_argmin(z_e, cb, W):
    B, T, D = z_e.shape
    N = B * T
    K = cb.shape[0]
    n_steps = N // _TILE
    t_per = T // _TILE
    return pl.pallas_call(
        _vq_body,
        grid=(n_steps,),
        in_specs=[
            pl.BlockSpec((1, _TILE, D), lambda i: (i // t_per, i % t_per, 0)),
            pl.BlockSpec((K, D), lambda i: (0, 0)),
            pl.BlockSpec((D, D), lambda i: (0, 0)),
        ],
        out_specs=[
            pl.BlockSpec((n_steps, 1, _TILE), lambda i: (0, 0, 0)),
            pl.BlockSpec((K, 128), lambda i: (0, 0)),
        ],
        out_shape=[
            jax.ShapeDtypeStruct((n_steps, 1, _TILE), jnp.int32),
            jax.ShapeDtypeStruct((K, 128), jnp.float32),
        ],
        scratch_shapes=[
            pltpu.VMEM((D + 6, K), jnp.float32),
        ],
    )(z_e, cb, W)


def _epilogue_body(z_ref, zq_pad_ref, idx_ref, zq_ref, loss_ref, ci_ref,
                   *, inv_nd):
    z = z_ref[...]
    b, t, d = z.shape
    zq = zq_pad_ref[:, 0:d].reshape(b, t, d)
    zq_ref[...] = zq
    diff = z - zq
    loss_ref[...] = (jnp.sum(diff * diff) * ((1.0 + _BETA) * inv_nd)
                     ).reshape(1, 1)
    ci_ref[...] = idx_ref[...].reshape(b, t)


def _slice_and_loss(z_e, zq_pad, idx3):
    B, T, D = z_e.shape
    body = functools.partial(_epilogue_body, inv_nd=1.0 / (B * T * D))
    return pl.pallas_call(
        body,
        out_shape=[
            jax.ShapeDtypeStruct((B, T, D), jnp.float32),
            jax.ShapeDtypeStruct((1, 1), jnp.float32),
            jax.ShapeDtypeStruct((B, T), jnp.int32),
        ],
    )(z_e, zq_pad, idx3)


def _sc_gather(eff, idx3):
    K, D = eff.shape
    n_steps, _, tile = idx3.shape
    N = n_steps * tile
    info = plsc.get_sparse_core_info()
    nw = info.num_cores * info.num_subcores
    per_w = N // nw
    w_per_step = tile // per_w
    mesh = plsc.VectorSubcoreMesh(core_axis_name="c", subcore_axis_name="s")

    @functools.partial(
        pl.kernel, mesh=mesh,
        out_type=jax.ShapeDtypeStruct((N, D), jnp.float32),
        scratch_types=[
            pltpu.VMEM((per_w,), jnp.int32),
            pltpu.VMEM((per_w, D), jnp.float32),
            pltpu.SemaphoreType.DMA,
        ],
    )
    def gather(table_hbm, idx_hbm, out_hbm, idx_v, rows_v, sem):
        wid = lax.axis_index("s") * info.num_cores + lax.axis_index("c")
        step = wid // w_per_step
        col = (wid % w_per_step) * per_w
        pltpu.sync_copy(idx_hbm.at[step, 0, pl.ds(col, per_w)], idx_v)
        pltpu.async_copy(table_hbm.at[idx_v], rows_v, sem).wait()
        pltpu.sync_copy(rows_v, out_hbm.at[pl.ds(wid * per_w, per_w)])

    return gather(eff, idx3)


def kernel(z_e, codebook, W):
    idx3, eff_pad = _distance_argmin(z_e, codebook, W)
    zq_pad = _sc_gather(eff_pad, idx3)
    z_q, loss, code_indices = _slice_and_loss(z_e, zq_pad, idx3)
    return z_q, loss[0, 0], code_indices
